# baseline (speedup 1.0000x reference)
import functools

import jax
import jax.numpy as jnp
import numpy as np
from jax import lax
from jax.experimental import pallas as pl
from jax.experimental.pallas import tpu as pltpu
from jax.experimental.pallas import tpu_sc as plsc

_EPS = 1e-5


def _fps_body(npoint, xyz_ref, c_ref):
    x = xyz_ref[:, 0, :]
    y = xyz_ref[:, 1, :]
    z = xyz_ref[:, 2, :]
    B, N = x.shape
    iota = lax.broadcasted_iota(jnp.int32, (B, N), 1)
    slot = lax.broadcasted_iota(jnp.int32, (1, npoint), 1)

    def body(i, carry):
        dist, far, ax, ay, az = carry
        sel = iota == far
        cx = jnp.sum(jnp.where(sel, x, 0.0), axis=1, keepdims=True)
        cy = jnp.sum(jnp.where(sel, y, 0.0), axis=1, keepdims=True)
        cz = jnp.sum(jnp.where(sel, z, 0.0), axis=1, keepdims=True)
        hit = slot == i
        ax = jnp.where(hit, cx, ax)
        ay = jnp.where(hit, cy, ay)
        az = jnp.where(hit, cz, az)
        dx = x - cx
        dy = y - cy
        dz = z - cz
        d = dx * dx + dy * dy + dz * dz
        dist = jnp.minimum(dist, d)
        m = jnp.max(dist, axis=1, keepdims=True)
        far = jnp.min(jnp.where(dist == m, iota, N), axis=1, keepdims=True)
        return dist, far, ax, ay, az

    zc = jnp.zeros((B, npoint), jnp.float32)
    _, _, ax, ay, az = lax.fori_loop(
        0, npoint, body,
        (jnp.full((B, N), 1e10, jnp.float32), jnp.zeros((B, 1), jnp.int32),
         zc, zc, zc))
    c_ref[:, 0, :] = ax
    c_ref[:, 1, :] = ay
    c_ref[:, 2, :] = az


def _fps(xyz, npoint):
    B, _, N = xyz.shape
    return pl.pallas_call(
        functools.partial(_fps_body, npoint),
        out_shape=jax.ShapeDtypeStruct((B, 3, npoint), jnp.float32),
    )(xyz)


def _bq_body(r2, nsample, xyz_ref, ct_ref, idx_ref):
    pts = xyz_ref[0]
    ct = ct_ref[0]
    S = ct.shape[0]
    N = pts.shape[1]
    cross = jnp.dot(ct, pts, preferred_element_type=jnp.float32)
    c2 = jnp.sum(ct * ct, axis=1, keepdims=True)
    p2 = jnp.sum(pts * pts, axis=0, keepdims=True)
    d = -2.0 * cross
    d = d + c2
    d = d + p2
    iota = lax.broadcasted_iota(jnp.int32, (S, N), 1)
    cand0 = jnp.where(d > r2, N, iota)
    slot = lax.broadcasted_iota(jnp.int32, (1, nsample), 1)

    def body(j, carry):
        cand, first, out = carry
        m = jnp.min(cand, axis=1, keepdims=True)
        first = jnp.where(j == 0, m, first)
        val = jnp.where(m == N, first, m)
        out = jnp.where(slot == j, val, out)
        cand = jnp.where(cand == m, N, cand)
        return cand, first, out

    _, _, out = lax.fori_loop(
        0, nsample, body,
        (cand0, jnp.zeros((S, 1), jnp.int32),
         jnp.zeros((S, nsample), jnp.int32)))
    idx_ref[0] = out


def _bq(radius, nsample, xyz, ct):
    B, _, N = xyz.shape
    S = ct.shape[1]
    r2 = np.float32(float(radius) ** 2)
    return pl.pallas_call(
        functools.partial(_bq_body, r2, nsample),
        grid=(B,),
        in_specs=[
            pl.BlockSpec((1, 3, N), lambda b: (b, 0, 0)),
            pl.BlockSpec((1, S, 3), lambda b: (b, 0, 0)),
        ],
        out_specs=pl.BlockSpec((1, S, nsample), lambda b: (b, 0, 0)),
        out_shape=jax.ShapeDtypeStruct((B, S, nsample), jnp.int32),
    )(xyz, ct)


def _sc_group(xyz, c, idxf, K):
    B, _, N = xyz.shape
    S = c.shape[2]
    R = S * K
    SH = S // 2
    NPT = SH * K
    shift = int(np.log2(K))
    mesh = plsc.VectorSubcoreMesh(core_axis_name="c", subcore_axis_name="s")
    xs = xyz[:, 0, :].reshape(B * N)
    ys = xyz[:, 1, :].reshape(B * N)
    zs = xyz[:, 2, :].reshape(B * N)
    cxs = c[:, 0, :].reshape(B * S)
    cys = c[:, 1, :].reshape(B * S)
    czs = c[:, 2, :].reshape(B * S)
    idxl = idxf.reshape(B * R)

    @functools.partial(
        pl.kernel,
        out_type=(jax.ShapeDtypeStruct((B * R,), jnp.float32),) * 3,
        mesh=mesh,
        compiler_params=pltpu.CompilerParams(needs_layout_passes=False),
        scratch_types=[
            pltpu.VMEM((N,), jnp.float32),
            pltpu.VMEM((N,), jnp.float32),
            pltpu.VMEM((N,), jnp.float32),
            pltpu.VMEM((SH,), jnp.float32),
            pltpu.VMEM((SH,), jnp.float32),
            pltpu.VMEM((SH,), jnp.float32),
            pltpu.VMEM((NPT,), jnp.int32),
            pltpu.VMEM((NPT,), jnp.float32),
            pltpu.VMEM((NPT,), jnp.float32),
            pltpu.VMEM((NPT,), jnp.float32),
        ],
    )
    def k(x_h, y_h, z_h, cx_h, cy_h, cz_h, idx_h, ox_h, oy_h, oz_h,
          xv, yv, zv, cxv, cyv, czv, idxv, bx, by, bz):
        wid = lax.axis_index("s") * 2 + lax.axis_index("c")
        b = wid // 2
        half = wid - 2 * b
        s0 = half * SH
        r0 = b * R + s0 * K
        pltpu.sync_copy(x_h.at[pl.ds(b * N, N)], xv)
        pltpu.sync_copy(y_h.at[pl.ds(b * N, N)], yv)
        pltpu.sync_copy(z_h.at[pl.ds(b * N, N)], zv)
        pltpu.sync_copy(cx_h.at[pl.ds(b * S + s0, SH)], cxv)
        pltpu.sync_copy(cy_h.at[pl.ds(b * S + s0, SH)], cyv)
        pltpu.sync_copy(cz_h.at[pl.ds(b * S + s0, SH)], czv)
        pltpu.sync_copy(idx_h.at[pl.ds(r0, NPT)], idxv)
        lane = lax.iota(jnp.int32, 16)

        def body(g, _):
            base = g * 16
            flat = base + lane
            sloc = jnp.right_shift(flat, shift)
            iv = idxv[pl.ds(base, 16)]
            bx[pl.ds(base, 16)] = (plsc.load_gather(xv, [iv])
                                   - plsc.load_gather(cxv, [sloc]))
            by[pl.ds(base, 16)] = (plsc.load_gather(yv, [iv])
                                   - plsc.load_gather(cyv, [sloc]))
            bz[pl.ds(base, 16)] = (plsc.load_gather(zv, [iv])
                                   - plsc.load_gather(czv, [sloc]))
            return 0

        lax.fori_loop(0, NPT // 16, body, 0)
        pltpu.sync_copy(bx, ox_h.at[pl.ds(r0, NPT)])
        pltpu.sync_copy(by, oy_h.at[pl.ds(r0, NPT)])
        pltpu.sync_copy(bz, oz_h.at[pl.ds(r0, NPT)])

    ox, oy, oz = k(xs, ys, zs, cxs, cys, czs, idxl)
    return ox.reshape(B, R), oy.reshape(B, R), oz.reshape(B, R)


def _mlp1_body(K, w1_ref, b1_ref, w2_ref, b2_ref, w3_ref, b3_ref,
               dx_ref, dy_ref, dz_ref, out_ref):
    w1 = w1_ref[...]
    dx = dx_ref[0]
    dy = dy_ref[0]
    dz = dz_ref[0]
    h = w1[:, 0:1] * dx + w1[:, 1:2] * dy + w1[:, 2:3] * dz + b1_ref[...]
    h = jnp.maximum(h, 0.0)
    h = jnp.dot(w2_ref[...], h, preferred_element_type=jnp.float32) + b2_ref[...]
    h = jnp.maximum(h, 0.0)
    h = jnp.dot(w3_ref[...], h, preferred_element_type=jnp.float32) + b3_ref[...]
    h = jnp.maximum(h, 0.0)
    C, R = h.shape
    out_ref[0] = jnp.max(h.reshape(C, R // K, K), axis=2)


def _mlp1(dx, dy, dz, K, w1, b1, w2, b2, w3, b3):
    B, R = dx.shape
    S = R // K
    C = w3.shape[0]
    dx = dx.reshape(B, 1, R)
    dy = dy.reshape(B, 1, R)
    dz = dz.reshape(B, 1, R)
    rspec = pl.BlockSpec((1, 1, R), lambda b: (b, 0, 0))
    wspec = lambda s: pl.BlockSpec(s, lambda b: tuple(0 for _ in s))
    return pl.pallas_call(
        functools.partial(_mlp1_body, K),
        grid=(B,),
        in_specs=[wspec(w1.shape), wspec(b1.shape), wspec(w2.shape),
                  wspec(b2.shape), wspec(w3.shape), wspec(b3.shape),
                  rspec, rspec, rspec],
        out_specs=pl.BlockSpec((1, C, S), lambda b: (b, 0, 0)),
        out_shape=jax.ShapeDtypeStruct((B, C, S), jnp.float32),
    )(w1, b1, w2, b2, w3, b3, dx, dy, dz)


def _mlp2_body(K, SB_S, w1x_ref, w1f_ref, b1_ref, w2_ref, b2_ref,
               w3_ref, b3_ref, idx_ref, xyz_ref, f_ref, c_ref, out_ref):
    ids = idx_ref[0, 0]
    xyzt = xyz_ref[0]
    feats = f_ref[0]
    N = xyzt.shape[1]
    R = ids.shape[1]
    onehot = (lax.broadcasted_iota(jnp.int32, (N, R), 0) == ids
              ).astype(jnp.float32)
    gx = jnp.dot(xyzt, onehot, preferred_element_type=jnp.float32)
    gf = jnp.dot(feats, onehot, preferred_element_type=jnp.float32)
    cc = c_ref[0, 0]
    crep = jnp.broadcast_to(cc[:, :, None], (3, SB_S, K)).reshape(3, R)
    dxyz = gx - crep
    h = (jnp.dot(w1x_ref[...], dxyz, preferred_element_type=jnp.float32)
         + jnp.dot(w1f_ref[...], gf, preferred_element_type=jnp.float32)
         + b1_ref[...])
    h = jnp.maximum(h, 0.0)
    h = jnp.dot(w2_ref[...], h, preferred_element_type=jnp.float32) + b2_ref[...]
    h = jnp.maximum(h, 0.0)
    h = jnp.dot(w3_ref[...], h, preferred_element_type=jnp.float32) + b3_ref[...]
    h = jnp.maximum(h, 0.0)
    C = h.shape[0]
    out_ref[0, 0] = jnp.max(h.reshape(C, SB_S, K), axis=2)


def _mlp2(idxf, xyzcm, featscm, ccm, K, w1x, w1f, b1, w2, b2, w3, b3):
    B, R = idxf.shape
    S = R // K
    N = xyzcm.shape[2]
    Cf = featscm.shape[1]
    C = w3.shape[0]
    SB = 4
    SB_S = S // SB
    RB = SB_S * K
    idxr = idxf.reshape(B, SB, 1, RB)
    ccr = ccm.reshape(B, 3, SB, SB_S).transpose(0, 2, 1, 3)
    wspec = lambda s: pl.BlockSpec(s, lambda b, sb: tuple(0 for _ in s))
    out = pl.pallas_call(
        functools.partial(_mlp2_body, K, SB_S),
        grid=(B, SB),
        in_specs=[wspec(w1x.shape), wspec(w1f.shape), wspec(b1.shape),
                  wspec(w2.shape), wspec(b2.shape), wspec(w3.shape),
                  wspec(b3.shape),
                  pl.BlockSpec((1, 1, 1, RB), lambda b, sb: (b, sb, 0, 0)),
                  pl.BlockSpec((1, 3, N), lambda b, sb: (b, 0, 0)),
                  pl.BlockSpec((1, Cf, N), lambda b, sb: (b, 0, 0)),
                  pl.BlockSpec((1, 1, 3, SB_S), lambda b, sb: (b, sb, 0, 0))],
        out_specs=pl.BlockSpec((1, 1, C, SB_S), lambda b, sb: (b, sb, 0, 0)),
        out_shape=jax.ShapeDtypeStruct((B, SB, C, SB_S), jnp.float32),
    )(w1x, w1f, b1, w2, b2, w3, b3, idxr, xyzcm, featscm, ccr)
    return out.transpose(0, 2, 1, 3).reshape(B, C, S)


def _sa3_body(w1x_ref, w1f_ref, b1_ref, w2_ref, b2_ref, w3_ref, b3_ref,
              f1w_ref, f1b_ref, f2w_ref, f2b_ref,
              xyz_ref, f_ref, l3_ref, x_ref):
    xyzp = xyz_ref[0]
    f = f_ref[0]
    h = (jnp.dot(w1x_ref[...], xyzp, preferred_element_type=jnp.float32)
         + jnp.dot(w1f_ref[...], f, preferred_element_type=jnp.float32)
         + b1_ref[...])
    h = jnp.maximum(h, 0.0)
    h = jnp.dot(w2_ref[...], h, preferred_element_type=jnp.float32) + b2_ref[...]
    h = jnp.maximum(h, 0.0)
    h = jnp.dot(w3_ref[...], h, preferred_element_type=jnp.float32) + b3_ref[...]
    h = jnp.maximum(h, 0.0)
    l3 = jnp.max(h, axis=1, keepdims=True)
    l3_ref[0] = l3
    y = jnp.dot(f1w_ref[...], l3, preferred_element_type=jnp.float32) + f1b_ref[...]
    y = jnp.maximum(y, 0.0)
    y = jnp.dot(f2w_ref[...], y, preferred_element_type=jnp.float32) + f2b_ref[...]
    y = jnp.maximum(y, 0.0)
    x_ref[0] = y


def _sa3_head(xyzcm, featscm, w1x, w1f, b1, w2, b2, w3, b3,
              f1w, f1b, f2w, f2b):
    B, Cf, S = featscm.shape
    wspec = lambda s: pl.BlockSpec(s, lambda b: tuple(0 for _ in s))
    return pl.pallas_call(
        _sa3_body,
        grid=(B,),
        in_specs=[wspec(w1x.shape), wspec(w1f.shape), wspec(b1.shape),
                  wspec(w2.shape), wspec(b2.shape), wspec(w3.shape),
                  wspec(b3.shape), wspec(f1w.shape), wspec(f1b.shape),
                  wspec(f2w.shape), wspec(f2b.shape),
                  pl.BlockSpec((1, 3, S), lambda b: (b, 0, 0)),
                  pl.BlockSpec((1, Cf, S), lambda b: (b, 0, 0))],
        out_specs=[pl.BlockSpec((1, 1024, 1), lambda b: (b, 0, 0)),
                   pl.BlockSpec((1, 256, 1), lambda b: (b, 0, 0))],
        out_shape=[jax.ShapeDtypeStruct((B, 1024, 1), jnp.float32),
                   jax.ShapeDtypeStruct((B, 256, 1), jnp.float32)],
    )(w1x, w1f, b1, w2, b2, w3, b3, f1w, f1b, f2w, f2b, xyzcm, featscm)


def _fold(p):
    s = p['g'] / jnp.sqrt(p['rv'] + _EPS)
    w = (p['W'] * s[None, :]).T
    b = ((p['b'] - p['rm']) * s + p['be'])[:, None]
    return w, b


def kernel(xyz, params):
    B, _, N = xyz.shape
    sa1 = [_fold(p) for p in params['sa1']]
    sa2 = [_fold(p) for p in params['sa2']]
    sa3 = [_fold(p) for p in params['sa3']]

    def _fold_fc(fc, bn):
        s = bn['g'] / jnp.sqrt(bn['rv'] + _EPS)
        w = (fc['W'] * s[None, :]).T
        b = ((fc['b'] - bn['rm']) * s + bn['be'])[:, None]
        return w, b

    f1w, f1b = _fold_fc(params['fc1'], params['bn1'])
    f2w, f2b = _fold_fc(params['fc2'], params['bn2'])

    c1 = _fps(xyz, 512)
    c1t = jnp.transpose(c1, (0, 2, 1))
    idx1 = _bq(0.2, 32, xyz, c1t)
    dx, dy, dz = _sc_group(xyz, c1, idx1.reshape(B, 512 * 32), 32)
    l1 = _mlp1(dx, dy, dz, 32,
               sa1[0][0], sa1[0][1], sa1[1][0], sa1[1][1],
               sa1[2][0], sa1[2][1])
    return l1[:, :, :256][:, 0], jnp.zeros((B, 1024, 1), jnp.float32)

    c2 = _fps(c1, 128)
    c2t = jnp.transpose(c2, (0, 2, 1))
    idx2 = _bq(0.4, 64, c1, c2t)
    w1 = sa2[0][0]
    l2 = _mlp2(idx2.reshape(B, 128 * 64), c1, l1, c2, 64,
               w1[:, :3], w1[:, 3:], sa2[0][1],
               sa2[1][0], sa2[1][1], sa2[2][0], sa2[2][1])

    w1g = sa3[0][0]
    l3, x = _sa3_head(c2, l2,
                      w1g[:, :3], w1g[:, 3:], sa3[0][1],
                      sa3[1][0], sa3[1][1], sa3[2][0], sa3[2][1],
                      f1w, f1b, f2w, f2b)
    return x.reshape(B, 256), l3


# speedup vs baseline: 1.1267x; 1.1267x over previous
import functools

import jax
import jax.numpy as jnp
import numpy as np
from jax import lax
from jax.experimental import pallas as pl
from jax.experimental.pallas import tpu as pltpu
from jax.experimental.pallas import tpu_sc as plsc

_EPS = 1e-5


def _fps_body(npoint, xyz_ref, c_ref):
    x = xyz_ref[:, 0, :]
    y = xyz_ref[:, 1, :]
    z = xyz_ref[:, 2, :]
    B, N = x.shape
    iota = lax.broadcasted_iota(jnp.int32, (B, N), 1)
    slot = lax.broadcasted_iota(jnp.int32, (1, npoint), 1)

    def body(i, carry):
        dist, far, ax, ay, az = carry
        sel = iota == far
        cx = jnp.sum(jnp.where(sel, x, 0.0), axis=1, keepdims=True)
        cy = jnp.sum(jnp.where(sel, y, 0.0), axis=1, keepdims=True)
        cz = jnp.sum(jnp.where(sel, z, 0.0), axis=1, keepdims=True)
        hit = slot == i
        ax = jnp.where(hit, cx, ax)
        ay = jnp.where(hit, cy, ay)
        az = jnp.where(hit, cz, az)
        dx = x - cx
        dy = y - cy
        dz = z - cz
        d = dx * dx + dy * dy + dz * dz
        dist = jnp.minimum(dist, d)
        m = jnp.max(dist, axis=1, keepdims=True)
        far = jnp.min(jnp.where(dist == m, iota, N), axis=1, keepdims=True)
        return dist, far, ax, ay, az

    zc = jnp.zeros((B, npoint), jnp.float32)
    _, _, ax, ay, az = lax.fori_loop(
        0, npoint, body,
        (jnp.full((B, N), 1e10, jnp.float32), jnp.zeros((B, 1), jnp.int32),
         zc, zc, zc))
    c_ref[:, 0, :] = ax
    c_ref[:, 1, :] = ay
    c_ref[:, 2, :] = az


def _fps(xyz, npoint):
    B, _, N = xyz.shape
    return pl.pallas_call(
        functools.partial(_fps_body, npoint),
        out_shape=jax.ShapeDtypeStruct((B, 3, npoint), jnp.float32),
    )(xyz)


def _bq_body(r2, nsample, xyz_ref, ct_ref, idx_ref):
    pts = xyz_ref[0]
    ct = ct_ref[0]
    S = ct.shape[0]
    N = pts.shape[1]
    cross = jnp.dot(ct, pts, preferred_element_type=jnp.float32)
    c2 = jnp.sum(ct * ct, axis=1, keepdims=True)
    p2 = jnp.sum(pts * pts, axis=0, keepdims=True)
    d = -2.0 * cross
    d = d + c2
    d = d + p2
    iota = lax.broadcasted_iota(jnp.int32, (S, N), 1)
    cand0 = jnp.where(d > r2, N, iota)
    slot = lax.broadcasted_iota(jnp.int32, (1, nsample), 1)

    def body(j, carry):
        cand, first, out = carry
        m = jnp.min(cand, axis=1, keepdims=True)
        first = jnp.where(j == 0, m, first)
        val = jnp.where(m == N, first, m)
        out = jnp.where(slot == j, val, out)
        cand = jnp.where(cand == m, N, cand)
        return cand, first, out

    _, _, out = lax.fori_loop(
        0, nsample, body,
        (cand0, jnp.zeros((S, 1), jnp.int32),
         jnp.zeros((S, nsample), jnp.int32)))
    idx_ref[0] = out


def _bq(radius, nsample, xyz, ct):
    B, _, N = xyz.shape
    S = ct.shape[1]
    r2 = np.float32(float(radius) ** 2)
    return pl.pallas_call(
        functools.partial(_bq_body, r2, nsample),
        grid=(B,),
        in_specs=[
            pl.BlockSpec((1, 3, N), lambda b: (b, 0, 0)),
            pl.BlockSpec((1, S, 3), lambda b: (b, 0, 0)),
        ],
        out_specs=pl.BlockSpec((1, S, nsample), lambda b: (b, 0, 0)),
        out_shape=jax.ShapeDtypeStruct((B, S, nsample), jnp.int32),
    )(xyz, ct)


def _bqbits_body(r2, xyz_ref, ct_ref, pk_ref, bits_ref):
    pts = xyz_ref[0]
    ct = ct_ref[0]
    cross = jnp.dot(ct, pts, preferred_element_type=jnp.float32)
    c2 = jnp.sum(ct * ct, axis=1, keepdims=True)
    p2 = jnp.sum(pts * pts, axis=0, keepdims=True)
    d = -2.0 * cross
    d = d + c2
    d = d + p2
    mask = (d <= r2).astype(jnp.float32)
    bits_ref[0] = jnp.dot(mask, pk_ref[...], preferred_element_type=jnp.float32)


def _bqbits(radius, xyz, ct):
    B, _, N = xyz.shape
    S = ct.shape[1]
    NW = N // 32
    r2 = np.float32(float(radius) ** 2)
    n = np.arange(N)
    pk = np.zeros((N, 2 * NW), np.float32)
    bit = n % 32
    lo = bit < 16
    pk[n[lo], n[lo] // 32] = (2.0 ** bit[lo]).astype(np.float32)
    pk[n[~lo], NW + n[~lo] // 32] = (2.0 ** (bit[~lo] - 16)).astype(np.float32)
    return pl.pallas_call(
        functools.partial(_bqbits_body, r2),
        grid=(B,),
        in_specs=[
            pl.BlockSpec((1, 3, N), lambda b: (b, 0, 0)),
            pl.BlockSpec((1, S, 3), lambda b: (b, 0, 0)),
            pl.BlockSpec((N, 2 * NW), lambda b: (0, 0)),
        ],
        out_specs=pl.BlockSpec((1, S, 2 * NW), lambda b: (b, 0, 0)),
        out_shape=jax.ShapeDtypeStruct((B, S, 2 * NW), jnp.float32),
    )(xyz, ct, jnp.asarray(pk))


def _sc_ballgather(xyz, c, bits, K):
    B, _, N = xyz.shape
    S = c.shape[2]
    R = S * K
    SH = S // 2
    NPT = SH * K
    NW = N // 32
    NG = NW // 16
    BW = 2 * NW
    shift = int(np.log2(K))
    mesh = plsc.VectorSubcoreMesh(core_axis_name="c", subcore_axis_name="s")
    xs = xyz[:, 0, :].reshape(B * N)
    ys = xyz[:, 1, :].reshape(B * N)
    zs = xyz[:, 2, :].reshape(B * N)
    cxs = c[:, 0, :].reshape(B * S)
    cys = c[:, 1, :].reshape(B * S)
    czs = c[:, 2, :].reshape(B * S)
    bitsl = bits.reshape(B * S * BW)

    @functools.partial(
        pl.kernel,
        out_type=(jax.ShapeDtypeStruct((B * R,), jnp.float32),) * 3,
        mesh=mesh,
        compiler_params=pltpu.CompilerParams(needs_layout_passes=False),
        scratch_types=[
            pltpu.VMEM((N,), jnp.float32),
            pltpu.VMEM((N,), jnp.float32),
            pltpu.VMEM((N,), jnp.float32),
            pltpu.VMEM((SH,), jnp.float32),
            pltpu.VMEM((SH,), jnp.float32),
            pltpu.VMEM((SH,), jnp.float32),
            pltpu.VMEM((SH * BW,), jnp.float32),
            pltpu.VMEM((NPT,), jnp.int32),
            pltpu.VMEM((NPT,), jnp.float32),
            pltpu.VMEM((NPT,), jnp.float32),
            pltpu.VMEM((NPT,), jnp.float32),
        ],
    )
    def k(x_h, y_h, z_h, cx_h, cy_h, cz_h, bits_h, ox_h, oy_h, oz_h,
          xv, yv, zv, cxv, cyv, czv, bv, idxb, bx, by, bz):
        wid = lax.axis_index("s") * 2 + lax.axis_index("c")
        b = wid // 2
        half = wid - 2 * b
        s0 = half * SH
        r0 = b * R + s0 * K
        pltpu.sync_copy(x_h.at[pl.ds(b * N, N)], xv)
        pltpu.sync_copy(y_h.at[pl.ds(b * N, N)], yv)
        pltpu.sync_copy(z_h.at[pl.ds(b * N, N)], zv)
        pltpu.sync_copy(cx_h.at[pl.ds(b * S + s0, SH)], cxv)
        pltpu.sync_copy(cy_h.at[pl.ds(b * S + s0, SH)], cyv)
        pltpu.sync_copy(cz_h.at[pl.ds(b * S + s0, SH)], czv)
        pltpu.sync_copy(bits_h.at[pl.ds((b * S + s0) * BW, SH * BW)], bv)
        lane = lax.iota(jnp.int32, 16)

        def row_body(rl, _):
            rowbase = rl * K
            bitbase = rl * BW

            def grp_cond(st):
                return jnp.logical_and(st[0] < NG, st[1] < K)

            def grp_body(st):
                g, cnt = st
                off = bitbase + g * 16
                lov = lax.convert_element_type(bv[pl.ds(off, 16)], jnp.int32)
                hiv = lax.convert_element_type(bv[pl.ds(off + NW, 16)],
                                               jnp.int32)
                w = jnp.bitwise_or(lov, jnp.left_shift(hiv, 16))
                v = w - (lax.shift_right_logical(w, 1) & 0x55555555)
                v = (v & 0x33333333) + (lax.shift_right_logical(v, 2)
                                        & 0x33333333)
                v = (v + lax.shift_right_logical(v, 4)) & 0x0F0F0F0F
                v = v + lax.shift_right_logical(v, 8)
                pc = (v + lax.shift_right_logical(v, 16)) & 0x3F
                csum = plsc.cumsum(pc)
                o = cnt + (csum - pc)
                nbase = (g * 16 + lane) * 32

                def ext_cond(st2):
                    return jnp.any(st2[0] != 0)

                def ext_body(st2):
                    ww, ext = st2
                    m = ww & (-ww)
                    f = lax.convert_element_type(m, jnp.float32)
                    e = (lax.shift_right_logical(
                        plsc.bitcast(f, jnp.int32), 23) & 0xFF) - 127
                    nz = ww != 0
                    pos = o + ext
                    plsc.store_scatter(idxb, [rowbase + pos], nbase + e,
                                       mask=jnp.logical_and(nz, pos < K))
                    return ww & (ww - 1), ext + nz.astype(jnp.int32)

                lax.while_loop(ext_cond, ext_body,
                               (w, jnp.zeros((16,), jnp.int32)))
                return g + 1, cnt + jnp.sum(pc)

            _, cnt = lax.while_loop(grp_cond, grp_body,
                                    (jnp.int32(0), jnp.int32(0)))
            f0 = plsc.load_gather(idxb, [jnp.zeros((16,), jnp.int32)
                                         + rowbase])
            for q in range(K // 16):
                sl = q * 16 + lane
                plsc.store_scatter(idxb, [rowbase + sl], f0, mask=sl >= cnt)
            return 0

        lax.fori_loop(0, SH, row_body, 0)

        def body(g, _):
            base = g * 16
            flat = base + lane
            sloc = jnp.right_shift(flat, shift)
            iv = idxb[pl.ds(base, 16)]
            bx[pl.ds(base, 16)] = (plsc.load_gather(xv, [iv])
                                   - plsc.load_gather(cxv, [sloc]))
            by[pl.ds(base, 16)] = (plsc.load_gather(yv, [iv])
                                   - plsc.load_gather(cyv, [sloc]))
            bz[pl.ds(base, 16)] = (plsc.load_gather(zv, [iv])
                                   - plsc.load_gather(czv, [sloc]))
            return 0

        lax.fori_loop(0, NPT // 16, body, 0)
        pltpu.sync_copy(bx, ox_h.at[pl.ds(r0, NPT)])
        pltpu.sync_copy(by, oy_h.at[pl.ds(r0, NPT)])
        pltpu.sync_copy(bz, oz_h.at[pl.ds(r0, NPT)])

    ox, oy, oz = k(xs, ys, zs, cxs, cys, czs, bitsl)
    return ox.reshape(B, R), oy.reshape(B, R), oz.reshape(B, R)


def _mlp1_body(K, w1_ref, b1_ref, w2_ref, b2_ref, w3_ref, b3_ref,
               dx_ref, dy_ref, dz_ref, out_ref):
    w1 = w1_ref[...]
    dx = dx_ref[0]
    dy = dy_ref[0]
    dz = dz_ref[0]
    h = w1[:, 0:1] * dx + w1[:, 1:2] * dy + w1[:, 2:3] * dz + b1_ref[...]
    h = jnp.maximum(h, 0.0)
    h = jnp.dot(w2_ref[...], h, preferred_element_type=jnp.float32) + b2_ref[...]
    h = jnp.maximum(h, 0.0)
    h = jnp.dot(w3_ref[...], h, preferred_element_type=jnp.float32) + b3_ref[...]
    h = jnp.maximum(h, 0.0)
    C, R = h.shape
    out_ref[0] = jnp.max(h.reshape(C, R // K, K), axis=2)


def _mlp1(dx, dy, dz, K, w1, b1, w2, b2, w3, b3):
    B, R = dx.shape
    S = R // K
    C = w3.shape[0]
    dx = dx.reshape(B, 1, R)
    dy = dy.reshape(B, 1, R)
    dz = dz.reshape(B, 1, R)
    rspec = pl.BlockSpec((1, 1, R), lambda b: (b, 0, 0))
    wspec = lambda s: pl.BlockSpec(s, lambda b: tuple(0 for _ in s))
    return pl.pallas_call(
        functools.partial(_mlp1_body, K),
        grid=(B,),
        in_specs=[wspec(w1.shape), wspec(b1.shape), wspec(w2.shape),
                  wspec(b2.shape), wspec(w3.shape), wspec(b3.shape),
                  rspec, rspec, rspec],
        out_specs=pl.BlockSpec((1, C, S), lambda b: (b, 0, 0)),
        out_shape=jax.ShapeDtypeStruct((B, C, S), jnp.float32),
    )(w1, b1, w2, b2, w3, b3, dx, dy, dz)


def _mlp2_body(K, SB_S, w1x_ref, w1f_ref, b1_ref, w2_ref, b2_ref,
               w3_ref, b3_ref, idx_ref, xyz_ref, f_ref, c_ref, out_ref):
    ids = idx_ref[0, 0]
    xyzt = xyz_ref[0]
    feats = f_ref[0]
    N = xyzt.shape[1]
    R = ids.shape[1]
    onehot = (lax.broadcasted_iota(jnp.int32, (N, R), 0) == ids
              ).astype(jnp.float32)
    gx = jnp.dot(xyzt, onehot, preferred_element_type=jnp.float32)
    gf = jnp.dot(feats, onehot, preferred_element_type=jnp.float32)
    cc = c_ref[0, 0]
    crep = jnp.broadcast_to(cc[:, :, None], (3, SB_S, K)).reshape(3, R)
    dxyz = gx - crep
    h = (jnp.dot(w1x_ref[...], dxyz, preferred_element_type=jnp.float32)
         + jnp.dot(w1f_ref[...], gf, preferred_element_type=jnp.float32)
         + b1_ref[...])
    h = jnp.maximum(h, 0.0)
    h = jnp.dot(w2_ref[...], h, preferred_element_type=jnp.float32) + b2_ref[...]
    h = jnp.maximum(h, 0.0)
    h = jnp.dot(w3_ref[...], h, preferred_element_type=jnp.float32) + b3_ref[...]
    h = jnp.maximum(h, 0.0)
    C = h.shape[0]
    out_ref[0, 0] = jnp.max(h.reshape(C, SB_S, K), axis=2)


def _mlp2(idxf, xyzcm, featscm, ccm, K, w1x, w1f, b1, w2, b2, w3, b3):
    B, R = idxf.shape
    S = R // K
    N = xyzcm.shape[2]
    Cf = featscm.shape[1]
    C = w3.shape[0]
    SB = 4
    SB_S = S // SB
    RB = SB_S * K
    idxr = idxf.reshape(B, SB, 1, RB)
    ccr = ccm.reshape(B, 3, SB, SB_S).transpose(0, 2, 1, 3)
    wspec = lambda s: pl.BlockSpec(s, lambda b, sb: tuple(0 for _ in s))
    out = pl.pallas_call(
        functools.partial(_mlp2_body, K, SB_S),
        grid=(B, SB),
        in_specs=[wspec(w1x.shape), wspec(w1f.shape), wspec(b1.shape),
                  wspec(w2.shape), wspec(b2.shape), wspec(w3.shape),
                  wspec(b3.shape),
                  pl.BlockSpec((1, 1, 1, RB), lambda b, sb: (b, sb, 0, 0)),
                  pl.BlockSpec((1, 3, N), lambda b, sb: (b, 0, 0)),
                  pl.BlockSpec((1, Cf, N), lambda b, sb: (b, 0, 0)),
                  pl.BlockSpec((1, 1, 3, SB_S), lambda b, sb: (b, sb, 0, 0))],
        out_specs=pl.BlockSpec((1, 1, C, SB_S), lambda b, sb: (b, sb, 0, 0)),
        out_shape=jax.ShapeDtypeStruct((B, SB, C, SB_S), jnp.float32),
    )(w1x, w1f, b1, w2, b2, w3, b3, idxr, xyzcm, featscm, ccr)
    return out.transpose(0, 2, 1, 3).reshape(B, C, S)


def _sa3_body(w1x_ref, w1f_ref, b1_ref, w2_ref, b2_ref, w3_ref, b3_ref,
              f1w_ref, f1b_ref, f2w_ref, f2b_ref,
              xyz_ref, f_ref, l3_ref, x_ref):
    xyzp = xyz_ref[0]
    f = f_ref[0]
    h = (jnp.dot(w1x_ref[...], xyzp, preferred_element_type=jnp.float32)
         + jnp.dot(w1f_ref[...], f, preferred_element_type=jnp.float32)
         + b1_ref[...])
    h = jnp.maximum(h, 0.0)
    h = jnp.dot(w2_ref[...], h, preferred_element_type=jnp.float32) + b2_ref[...]
    h = jnp.maximum(h, 0.0)
    h = jnp.dot(w3_ref[...], h, preferred_element_type=jnp.float32) + b3_ref[...]
    h = jnp.maximum(h, 0.0)
    l3 = jnp.max(h, axis=1, keepdims=True)
    l3_ref[0] = l3
    y = jnp.dot(f1w_ref[...], l3, preferred_element_type=jnp.float32) + f1b_ref[...]
    y = jnp.maximum(y, 0.0)
    y = jnp.dot(f2w_ref[...], y, preferred_element_type=jnp.float32) + f2b_ref[...]
    y = jnp.maximum(y, 0.0)
    x_ref[0] = y


def _sa3_head(xyzcm, featscm, w1x, w1f, b1, w2, b2, w3, b3,
              f1w, f1b, f2w, f2b):
    B, Cf, S = featscm.shape
    wspec = lambda s: pl.BlockSpec(s, lambda b: tuple(0 for _ in s))
    return pl.pallas_call(
        _sa3_body,
        grid=(B,),
        in_specs=[wspec(w1x.shape), wspec(w1f.shape), wspec(b1.shape),
                  wspec(w2.shape), wspec(b2.shape), wspec(w3.shape),
                  wspec(b3.shape), wspec(f1w.shape), wspec(f1b.shape),
                  wspec(f2w.shape), wspec(f2b.shape),
                  pl.BlockSpec((1, 3, S), lambda b: (b, 0, 0)),
                  pl.BlockSpec((1, Cf, S), lambda b: (b, 0, 0))],
        out_specs=[pl.BlockSpec((1, 1024, 1), lambda b: (b, 0, 0)),
                   pl.BlockSpec((1, 256, 1), lambda b: (b, 0, 0))],
        out_shape=[jax.ShapeDtypeStruct((B, 1024, 1), jnp.float32),
                   jax.ShapeDtypeStruct((B, 256, 1), jnp.float32)],
    )(w1x, w1f, b1, w2, b2, w3, b3, f1w, f1b, f2w, f2b, xyzcm, featscm)


def _fold(p):
    s = p['g'] / jnp.sqrt(p['rv'] + _EPS)
    w = (p['W'] * s[None, :]).T
    b = ((p['b'] - p['rm']) * s + p['be'])[:, None]
    return w, b


def kernel(xyz, params):
    B, _, N = xyz.shape
    sa1 = [_fold(p) for p in params['sa1']]
    sa2 = [_fold(p) for p in params['sa2']]
    sa3 = [_fold(p) for p in params['sa3']]

    def _fold_fc(fc, bn):
        s = bn['g'] / jnp.sqrt(bn['rv'] + _EPS)
        w = (fc['W'] * s[None, :]).T
        b = ((fc['b'] - bn['rm']) * s + bn['be'])[:, None]
        return w, b

    f1w, f1b = _fold_fc(params['fc1'], params['bn1'])
    f2w, f2b = _fold_fc(params['fc2'], params['bn2'])

    c1 = _fps(xyz, 512)
    c1t = jnp.transpose(c1, (0, 2, 1))
    bits1 = _bqbits(0.2, xyz, c1t)
    dx, dy, dz = _sc_ballgather(xyz, c1, bits1, 32)
    l1 = _mlp1(dx, dy, dz, 32,
               sa1[0][0], sa1[0][1], sa1[1][0], sa1[1][1],
               sa1[2][0], sa1[2][1])

    c2 = _fps(c1, 128)
    c2t = jnp.transpose(c2, (0, 2, 1))
    idx2 = _bq(0.4, 64, c1, c2t)
    w1 = sa2[0][0]
    l2 = _mlp2(idx2.reshape(B, 128 * 64), c1, l1, c2, 64,
               w1[:, :3], w1[:, 3:], sa2[0][1],
               sa2[1][0], sa2[1][1], sa2[2][0], sa2[2][1])

    w1g = sa3[0][0]
    l3, x = _sa3_head(c2, l2,
                      w1g[:, :3], w1g[:, 3:], sa3[0][1],
                      sa3[1][0], sa3[1][1], sa3[2][0], sa3[2][1],
                      f1w, f1b, f2w, f2b)
    return x.reshape(B, 256), l3


# speedup vs baseline: 1.3660x; 1.2124x over previous
import functools

import jax
import jax.numpy as jnp
import numpy as np
from jax import lax
from jax.experimental import pallas as pl
from jax.experimental.pallas import tpu as pltpu
from jax.experimental.pallas import tpu_sc as plsc

_EPS = 1e-5


def _fps_body(npoint, xyz_ref, c_ref):
    x = xyz_ref[:, 0, :]
    y = xyz_ref[:, 1, :]
    z = xyz_ref[:, 2, :]
    B, N = x.shape
    iota = lax.broadcasted_iota(jnp.int32, (B, N), 1)
    slot = lax.broadcasted_iota(jnp.int32, (1, npoint), 1)

    def body(i, carry):
        dist, far, ax, ay, az = carry
        sel = iota == far
        cx = jnp.sum(jnp.where(sel, x, 0.0), axis=1, keepdims=True)
        cy = jnp.sum(jnp.where(sel, y, 0.0), axis=1, keepdims=True)
        cz = jnp.sum(jnp.where(sel, z, 0.0), axis=1, keepdims=True)
        hit = slot == i
        ax = jnp.where(hit, cx, ax)
        ay = jnp.where(hit, cy, ay)
        az = jnp.where(hit, cz, az)
        dx = x - cx
        dy = y - cy
        dz = z - cz
        d = dx * dx + dy * dy + dz * dz
        dist = jnp.minimum(dist, d)
        m = jnp.max(dist, axis=1, keepdims=True)
        far = jnp.min(jnp.where(dist == m, iota, N), axis=1, keepdims=True)
        return dist, far, ax, ay, az

    zc = jnp.zeros((B, npoint), jnp.float32)
    _, _, ax, ay, az = lax.fori_loop(
        0, npoint, body,
        (jnp.full((B, N), 1e10, jnp.float32), jnp.zeros((B, 1), jnp.int32),
         zc, zc, zc))
    c_ref[:, 0, :] = ax
    c_ref[:, 1, :] = ay
    c_ref[:, 2, :] = az


def _fps(xyz, npoint):
    B, _, N = xyz.shape
    return pl.pallas_call(
        functools.partial(_fps_body, npoint),
        out_shape=jax.ShapeDtypeStruct((B, 3, npoint), jnp.float32),
    )(xyz)


def _bq_body(r2, nsample, xyz_ref, ct_ref, idx_ref):
    pts = xyz_ref[0]
    ct = ct_ref[0]
    S = ct.shape[0]
    N = pts.shape[1]
    cross = jnp.dot(ct, pts, preferred_element_type=jnp.float32)
    c2 = jnp.sum(ct * ct, axis=1, keepdims=True)
    p2 = jnp.sum(pts * pts, axis=0, keepdims=True)
    d = -2.0 * cross
    d = d + c2
    d = d + p2
    iota = lax.broadcasted_iota(jnp.int32, (S, N), 1)
    cand0 = jnp.where(d > r2, N, iota)
    slot = lax.broadcasted_iota(jnp.int32, (1, nsample), 1)

    def body(j, carry):
        cand, first, out = carry
        m = jnp.min(cand, axis=1, keepdims=True)
        first = jnp.where(j == 0, m, first)
        val = jnp.where(m == N, first, m)
        out = jnp.where(slot == j, val, out)
        cand = jnp.where(cand == m, N, cand)
        return cand, first, out

    _, _, out = lax.fori_loop(
        0, nsample, body,
        (cand0, jnp.zeros((S, 1), jnp.int32),
         jnp.zeros((S, nsample), jnp.int32)))
    idx_ref[0] = out


def _bq(radius, nsample, xyz, ct):
    B, _, N = xyz.shape
    S = ct.shape[1]
    r2 = np.float32(float(radius) ** 2)
    return pl.pallas_call(
        functools.partial(_bq_body, r2, nsample),
        grid=(B,),
        in_specs=[
            pl.BlockSpec((1, 3, N), lambda b: (b, 0, 0)),
            pl.BlockSpec((1, S, 3), lambda b: (b, 0, 0)),
        ],
        out_specs=pl.BlockSpec((1, S, nsample), lambda b: (b, 0, 0)),
        out_shape=jax.ShapeDtypeStruct((B, S, nsample), jnp.int32),
    )(xyz, ct)


def _bqbits_body(r2, xyz_ref, ct_ref, pk_ref, bits_ref):
    pts = xyz_ref[0]
    ct = ct_ref[0]
    cross = jnp.dot(ct, pts, preferred_element_type=jnp.float32)
    c2 = jnp.sum(ct * ct, axis=1, keepdims=True)
    p2 = jnp.sum(pts * pts, axis=0, keepdims=True)
    d = -2.0 * cross
    d = d + c2
    d = d + p2
    mask = (d <= r2).astype(jnp.float32)
    bits_ref[0] = jnp.dot(mask, pk_ref[...], preferred_element_type=jnp.float32)


def _bqbits(radius, xyz, ct):
    B, _, N = xyz.shape
    S = ct.shape[1]
    NW = N // 32
    r2 = np.float32(float(radius) ** 2)
    n = np.arange(N)
    pk = np.zeros((N, 2 * NW), np.float32)
    bit = n % 32
    lo = bit < 16
    pk[n[lo], n[lo] // 32] = (2.0 ** bit[lo]).astype(np.float32)
    pk[n[~lo], NW + n[~lo] // 32] = (2.0 ** (bit[~lo] - 16)).astype(np.float32)
    return pl.pallas_call(
        functools.partial(_bqbits_body, r2),
        grid=(B,),
        in_specs=[
            pl.BlockSpec((1, 3, N), lambda b: (b, 0, 0)),
            pl.BlockSpec((1, S, 3), lambda b: (b, 0, 0)),
            pl.BlockSpec((N, 2 * NW), lambda b: (0, 0)),
        ],
        out_specs=pl.BlockSpec((1, S, 2 * NW), lambda b: (b, 0, 0)),
        out_shape=jax.ShapeDtypeStruct((B, S, 2 * NW), jnp.float32),
    )(xyz, ct, jnp.asarray(pk))


def _sc_ballgather(xyz, c, bits, K):
    B, _, N = xyz.shape
    S = c.shape[2]
    R = S * K
    SH = S // 2
    NPT = SH * K
    NW = N // 32
    NG = NW // 16
    BW = 2 * NW
    shift = int(np.log2(K))
    mesh = plsc.VectorSubcoreMesh(core_axis_name="c", subcore_axis_name="s")
    xs = xyz[:, 0, :].reshape(B * N)
    ys = xyz[:, 1, :].reshape(B * N)
    zs = xyz[:, 2, :].reshape(B * N)
    cxs = c[:, 0, :].reshape(B * S)
    cys = c[:, 1, :].reshape(B * S)
    czs = c[:, 2, :].reshape(B * S)
    bitsl = bits.reshape(B * S * BW)

    @functools.partial(
        pl.kernel,
        out_type=(jax.ShapeDtypeStruct((B * R,), jnp.float32),) * 3,
        mesh=mesh,
        compiler_params=pltpu.CompilerParams(needs_layout_passes=False),
        scratch_types=[
            pltpu.VMEM((N,), jnp.float32),
            pltpu.VMEM((N,), jnp.float32),
            pltpu.VMEM((N,), jnp.float32),
            pltpu.VMEM((SH,), jnp.float32),
            pltpu.VMEM((SH,), jnp.float32),
            pltpu.VMEM((SH,), jnp.float32),
            pltpu.VMEM((SH * BW,), jnp.float32),
            pltpu.VMEM((NPT,), jnp.int32),
            pltpu.VMEM((NPT,), jnp.float32),
            pltpu.VMEM((NPT,), jnp.float32),
            pltpu.VMEM((NPT,), jnp.float32),
        ],
    )
    def k(x_h, y_h, z_h, cx_h, cy_h, cz_h, bits_h, ox_h, oy_h, oz_h,
          xv, yv, zv, cxv, cyv, czv, bv, idxb, bx, by, bz):
        wid = lax.axis_index("s") * 2 + lax.axis_index("c")
        b = wid // 2
        half = wid - 2 * b
        s0 = half * SH
        r0 = b * R + s0 * K
        pltpu.sync_copy(x_h.at[pl.ds(b * N, N)], xv)
        pltpu.sync_copy(y_h.at[pl.ds(b * N, N)], yv)
        pltpu.sync_copy(z_h.at[pl.ds(b * N, N)], zv)
        pltpu.sync_copy(cx_h.at[pl.ds(b * S + s0, SH)], cxv)
        pltpu.sync_copy(cy_h.at[pl.ds(b * S + s0, SH)], cyv)
        pltpu.sync_copy(cz_h.at[pl.ds(b * S + s0, SH)], czv)
        pltpu.sync_copy(bits_h.at[pl.ds((b * S + s0) * BW, SH * BW)], bv)
        lane = lax.iota(jnp.int32, 16)

        def row_body(rl, _):
            rowbase = rl * K
            bitbase = rl * BW

            def grp_cond(st):
                return jnp.logical_and(st[0] < NG, st[1] < K)

            def grp_body(st):
                g, cnt = st
                off = bitbase + g * 16
                lov = lax.convert_element_type(bv[pl.ds(off, 16)], jnp.int32)
                hiv = lax.convert_element_type(bv[pl.ds(off + NW, 16)],
                                               jnp.int32)
                w = jnp.bitwise_or(lov, jnp.left_shift(hiv, 16))
                v = w - (lax.shift_right_logical(w, 1) & 0x55555555)
                v = (v & 0x33333333) + (lax.shift_right_logical(v, 2)
                                        & 0x33333333)
                v = (v + lax.shift_right_logical(v, 4)) & 0x0F0F0F0F
                v = v + lax.shift_right_logical(v, 8)
                pc = (v + lax.shift_right_logical(v, 16)) & 0x3F
                csum = plsc.cumsum(pc)
                o = cnt + (csum - pc)
                nbase = (g * 16 + lane) * 32

                def ext_cond(st2):
                    return jnp.any(st2[0] != 0)

                def ext_body(st2):
                    ww, ext = st2
                    m = ww & (-ww)
                    f = lax.convert_element_type(m, jnp.float32)
                    e = (lax.shift_right_logical(
                        plsc.bitcast(f, jnp.int32), 23) & 0xFF) - 127
                    nz = ww != 0
                    pos = o + ext
                    plsc.store_scatter(idxb, [rowbase + pos], nbase + e,
                                       mask=jnp.logical_and(nz, pos < K))
                    return ww & (ww - 1), ext + nz.astype(jnp.int32)

                lax.while_loop(ext_cond, ext_body,
                               (w, jnp.zeros((16,), jnp.int32)))
                return g + 1, cnt + jnp.sum(pc)

            _, cnt = lax.while_loop(grp_cond, grp_body,
                                    (jnp.int32(0), jnp.int32(0)))
            f0 = plsc.load_gather(idxb, [jnp.zeros((16,), jnp.int32)
                                         + rowbase])
            for q in range(K // 16):
                sl = q * 16 + lane
                plsc.store_scatter(idxb, [rowbase + sl], f0, mask=sl >= cnt)
            return 0

        lax.fori_loop(0, SH, row_body, 0)

        def body(g, _):
            base = g * 16
            flat = base + lane
            sloc = jnp.right_shift(flat, shift)
            iv = idxb[pl.ds(base, 16)]
            bx[pl.ds(base, 16)] = (plsc.load_gather(xv, [iv])
                                   - plsc.load_gather(cxv, [sloc]))
            by[pl.ds(base, 16)] = (plsc.load_gather(yv, [iv])
                                   - plsc.load_gather(cyv, [sloc]))
            bz[pl.ds(base, 16)] = (plsc.load_gather(zv, [iv])
                                   - plsc.load_gather(czv, [sloc]))
            return 0

        lax.fori_loop(0, NPT // 16, body, 0)
        pltpu.sync_copy(bx, ox_h.at[pl.ds(r0, NPT)])
        pltpu.sync_copy(by, oy_h.at[pl.ds(r0, NPT)])
        pltpu.sync_copy(bz, oz_h.at[pl.ds(r0, NPT)])

    ox, oy, oz = k(xs, ys, zs, cxs, cys, czs, bitsl)
    return ox.reshape(B, R), oy.reshape(B, R), oz.reshape(B, R)


def _sc_ballidx(bits, K):
    B, S, BW = bits.shape
    NW = BW // 2
    NG = NW // 16
    R = S * K
    SH = (B * S) // 32
    NPT = SH * K
    mesh = plsc.VectorSubcoreMesh(core_axis_name="c", subcore_axis_name="s")
    bitsl = bits.reshape(B * S * BW)

    @functools.partial(
        pl.kernel,
        out_type=jax.ShapeDtypeStruct((B * R,), jnp.int32),
        mesh=mesh,
        compiler_params=pltpu.CompilerParams(needs_layout_passes=False),
        scratch_types=[
            pltpu.VMEM((SH * BW,), jnp.float32),
            pltpu.VMEM((NPT,), jnp.int32),
        ],
    )
    def k(bits_h, oi_h, bv, idxb):
        wid = lax.axis_index("s") * 2 + lax.axis_index("c")
        row0 = wid * SH
        pltpu.sync_copy(bits_h.at[pl.ds(row0 * BW, SH * BW)], bv)
        lane = lax.iota(jnp.int32, 16)

        def row_body(rl, _):
            rowbase = rl * K
            bitbase = rl * BW

            def grp_cond(st):
                return jnp.logical_and(st[0] < NG, st[1] < K)

            def grp_body(st):
                g, cnt = st
                off = bitbase + g * 16
                lov = lax.convert_element_type(bv[pl.ds(off, 16)], jnp.int32)
                hiv = lax.convert_element_type(bv[pl.ds(off + NW, 16)],
                                               jnp.int32)
                w = jnp.bitwise_or(lov, jnp.left_shift(hiv, 16))
                v = w - (lax.shift_right_logical(w, 1) & 0x55555555)
                v = (v & 0x33333333) + (lax.shift_right_logical(v, 2)
                                        & 0x33333333)
                v = (v + lax.shift_right_logical(v, 4)) & 0x0F0F0F0F
                v = v + lax.shift_right_logical(v, 8)
                pc = (v + lax.shift_right_logical(v, 16)) & 0x3F
                csum = plsc.cumsum(pc)
                o = cnt + (csum - pc)
                nbase = (g * 16 + lane) * 32

                def ext_cond(st2):
                    return jnp.any(st2[0] != 0)

                def ext_body(st2):
                    ww, ext = st2
                    m = ww & (-ww)
                    f = lax.convert_element_type(m, jnp.float32)
                    e = (lax.shift_right_logical(
                        plsc.bitcast(f, jnp.int32), 23) & 0xFF) - 127
                    nz = ww != 0
                    pos = o + ext
                    plsc.store_scatter(idxb, [rowbase + pos], nbase + e,
                                       mask=jnp.logical_and(nz, pos < K))
                    return ww & (ww - 1), ext + nz.astype(jnp.int32)

                lax.while_loop(ext_cond, ext_body,
                               (w, jnp.zeros((16,), jnp.int32)))
                return g + 1, cnt + jnp.sum(pc)

            _, cnt = lax.while_loop(grp_cond, grp_body,
                                    (jnp.int32(0), jnp.int32(0)))
            f0 = plsc.load_gather(idxb, [jnp.zeros((16,), jnp.int32)
                                         + rowbase])
            for q in range(K // 16):
                sl = q * 16 + lane
                plsc.store_scatter(idxb, [rowbase + sl], f0, mask=sl >= cnt)
            return 0

        lax.fori_loop(0, SH, row_body, 0)
        pltpu.sync_copy(idxb, oi_h.at[pl.ds(row0 * K, NPT)])

    return k(bitsl).reshape(B, R)


def _mlp1_body(K, w1_ref, b1_ref, w2_ref, b2_ref, w3_ref, b3_ref,
               dx_ref, dy_ref, dz_ref, out_ref):
    w1 = w1_ref[...]
    dx = dx_ref[0]
    dy = dy_ref[0]
    dz = dz_ref[0]
    h = w1[:, 0:1] * dx + w1[:, 1:2] * dy + w1[:, 2:3] * dz + b1_ref[...]
    h = jnp.maximum(h, 0.0)
    h = jnp.dot(w2_ref[...], h, preferred_element_type=jnp.float32) + b2_ref[...]
    h = jnp.maximum(h, 0.0)
    h = jnp.dot(w3_ref[...], h, preferred_element_type=jnp.float32) + b3_ref[...]
    h = jnp.maximum(h, 0.0)
    C, R = h.shape
    out_ref[0] = jnp.max(h.reshape(C, R // K, K), axis=2)


def _mlp1(dx, dy, dz, K, w1, b1, w2, b2, w3, b3):
    B, R = dx.shape
    S = R // K
    C = w3.shape[0]
    dx = dx.reshape(B, 1, R)
    dy = dy.reshape(B, 1, R)
    dz = dz.reshape(B, 1, R)
    rspec = pl.BlockSpec((1, 1, R), lambda b: (b, 0, 0))
    wspec = lambda s: pl.BlockSpec(s, lambda b: tuple(0 for _ in s))
    return pl.pallas_call(
        functools.partial(_mlp1_body, K),
        grid=(B,),
        in_specs=[wspec(w1.shape), wspec(b1.shape), wspec(w2.shape),
                  wspec(b2.shape), wspec(w3.shape), wspec(b3.shape),
                  rspec, rspec, rspec],
        out_specs=pl.BlockSpec((1, C, S), lambda b: (b, 0, 0)),
        out_shape=jax.ShapeDtypeStruct((B, C, S), jnp.float32),
    )(w1, b1, w2, b2, w3, b3, dx, dy, dz)


def _mlp2_body(K, SB_S, w1x_ref, w1f_ref, b1_ref, w2_ref, b2_ref,
               w3_ref, b3_ref, idx_ref, xyz_ref, f_ref, c_ref, out_ref):
    ids = idx_ref[0, 0]
    xyzt = xyz_ref[0]
    feats = f_ref[0]
    N = xyzt.shape[1]
    R = ids.shape[1]
    onehot = (lax.broadcasted_iota(jnp.int32, (N, R), 0) == ids
              ).astype(jnp.float32)
    gx = jnp.dot(xyzt, onehot, preferred_element_type=jnp.float32)
    gf = jnp.dot(feats, onehot, preferred_element_type=jnp.float32)
    cc = c_ref[0, 0]
    crep = jnp.broadcast_to(cc[:, :, None], (3, SB_S, K)).reshape(3, R)
    dxyz = gx - crep
    h = (jnp.dot(w1x_ref[...], dxyz, preferred_element_type=jnp.float32)
         + jnp.dot(w1f_ref[...], gf, preferred_element_type=jnp.float32)
         + b1_ref[...])
    h = jnp.maximum(h, 0.0)
    h = jnp.dot(w2_ref[...], h, preferred_element_type=jnp.float32) + b2_ref[...]
    h = jnp.maximum(h, 0.0)
    h = jnp.dot(w3_ref[...], h, preferred_element_type=jnp.float32) + b3_ref[...]
    h = jnp.maximum(h, 0.0)
    C = h.shape[0]
    out_ref[0, 0] = jnp.max(h.reshape(C, SB_S, K), axis=2)


def _mlp2(idxf, xyzcm, featscm, ccm, K, w1x, w1f, b1, w2, b2, w3, b3):
    B, R = idxf.shape
    S = R // K
    N = xyzcm.shape[2]
    Cf = featscm.shape[1]
    C = w3.shape[0]
    SB = 4
    SB_S = S // SB
    RB = SB_S * K
    idxr = idxf.reshape(B, SB, 1, RB)
    ccr = ccm.reshape(B, 3, SB, SB_S).transpose(0, 2, 1, 3)
    wspec = lambda s: pl.BlockSpec(s, lambda b, sb: tuple(0 for _ in s))
    out = pl.pallas_call(
        functools.partial(_mlp2_body, K, SB_S),
        grid=(B, SB),
        in_specs=[wspec(w1x.shape), wspec(w1f.shape), wspec(b1.shape),
                  wspec(w2.shape), wspec(b2.shape), wspec(w3.shape),
                  wspec(b3.shape),
                  pl.BlockSpec((1, 1, 1, RB), lambda b, sb: (b, sb, 0, 0)),
                  pl.BlockSpec((1, 3, N), lambda b, sb: (b, 0, 0)),
                  pl.BlockSpec((1, Cf, N), lambda b, sb: (b, 0, 0)),
                  pl.BlockSpec((1, 1, 3, SB_S), lambda b, sb: (b, sb, 0, 0))],
        out_specs=pl.BlockSpec((1, 1, C, SB_S), lambda b, sb: (b, sb, 0, 0)),
        out_shape=jax.ShapeDtypeStruct((B, SB, C, SB_S), jnp.float32),
    )(w1x, w1f, b1, w2, b2, w3, b3, idxr, xyzcm, featscm, ccr)
    return out.transpose(0, 2, 1, 3).reshape(B, C, S)


def _sa3_body(w1x_ref, w1f_ref, b1_ref, w2_ref, b2_ref, w3_ref, b3_ref,
              f1w_ref, f1b_ref, f2w_ref, f2b_ref,
              xyz_ref, f_ref, l3_ref, x_ref):
    xyzp = xyz_ref[0]
    f = f_ref[0]
    h = (jnp.dot(w1x_ref[...], xyzp, preferred_element_type=jnp.float32)
         + jnp.dot(w1f_ref[...], f, preferred_element_type=jnp.float32)
         + b1_ref[...])
    h = jnp.maximum(h, 0.0)
    h = jnp.dot(w2_ref[...], h, preferred_element_type=jnp.float32) + b2_ref[...]
    h = jnp.maximum(h, 0.0)
    h = jnp.dot(w3_ref[...], h, preferred_element_type=jnp.float32) + b3_ref[...]
    h = jnp.maximum(h, 0.0)
    l3 = jnp.max(h, axis=1, keepdims=True)
    l3_ref[0] = l3
    y = jnp.dot(f1w_ref[...], l3, preferred_element_type=jnp.float32) + f1b_ref[...]
    y = jnp.maximum(y, 0.0)
    y = jnp.dot(f2w_ref[...], y, preferred_element_type=jnp.float32) + f2b_ref[...]
    y = jnp.maximum(y, 0.0)
    x_ref[0] = y


def _sa3_head(xyzcm, featscm, w1x, w1f, b1, w2, b2, w3, b3,
              f1w, f1b, f2w, f2b):
    B, Cf, S = featscm.shape
    wspec = lambda s: pl.BlockSpec(s, lambda b: tuple(0 for _ in s))
    return pl.pallas_call(
        _sa3_body,
        grid=(B,),
        in_specs=[wspec(w1x.shape), wspec(w1f.shape), wspec(b1.shape),
                  wspec(w2.shape), wspec(b2.shape), wspec(w3.shape),
                  wspec(b3.shape), wspec(f1w.shape), wspec(f1b.shape),
                  wspec(f2w.shape), wspec(f2b.shape),
                  pl.BlockSpec((1, 3, S), lambda b: (b, 0, 0)),
                  pl.BlockSpec((1, Cf, S), lambda b: (b, 0, 0))],
        out_specs=[pl.BlockSpec((1, 1024, 1), lambda b: (b, 0, 0)),
                   pl.BlockSpec((1, 256, 1), lambda b: (b, 0, 0))],
        out_shape=[jax.ShapeDtypeStruct((B, 1024, 1), jnp.float32),
                   jax.ShapeDtypeStruct((B, 256, 1), jnp.float32)],
    )(w1x, w1f, b1, w2, b2, w3, b3, f1w, f1b, f2w, f2b, xyzcm, featscm)


def _fold(p):
    s = p['g'] / jnp.sqrt(p['rv'] + _EPS)
    w = (p['W'] * s[None, :]).T
    b = ((p['b'] - p['rm']) * s + p['be'])[:, None]
    return w, b


def kernel(xyz, params):
    B, _, N = xyz.shape
    sa1 = [_fold(p) for p in params['sa1']]
    sa2 = [_fold(p) for p in params['sa2']]
    sa3 = [_fold(p) for p in params['sa3']]

    def _fold_fc(fc, bn):
        s = bn['g'] / jnp.sqrt(bn['rv'] + _EPS)
        w = (fc['W'] * s[None, :]).T
        b = ((fc['b'] - bn['rm']) * s + bn['be'])[:, None]
        return w, b

    f1w, f1b = _fold_fc(params['fc1'], params['bn1'])
    f2w, f2b = _fold_fc(params['fc2'], params['bn2'])

    c1 = _fps(xyz, 512)
    c1t = jnp.transpose(c1, (0, 2, 1))
    bits1 = _bqbits(0.2, xyz, c1t)
    dx, dy, dz = _sc_ballgather(xyz, c1, bits1, 32)
    l1 = _mlp1(dx, dy, dz, 32,
               sa1[0][0], sa1[0][1], sa1[1][0], sa1[1][1],
               sa1[2][0], sa1[2][1])

    c2 = _fps(c1, 128)
    c2t = jnp.transpose(c2, (0, 2, 1))
    bits2 = _bqbits(0.4, c1, c2t)
    idx2 = _sc_ballidx(bits2, 64)
    w1 = sa2[0][0]
    l2 = _mlp2(idx2, c1, l1, c2, 64,
               w1[:, :3], w1[:, 3:], sa2[0][1],
               sa2[1][0], sa2[1][1], sa2[2][0], sa2[2][1])

    w1g = sa3[0][0]
    l3, x = _sa3_head(c2, l2,
                      w1g[:, :3], w1g[:, 3:], sa3[0][1],
                      sa3[1][0], sa3[1][1], sa3[2][0], sa3[2][1],
                      f1w, f1b, f2w, f2b)
    return x.reshape(B, 256), l3


# speedup vs baseline: 1.5779x; 1.1551x over previous
import functools

import jax
import jax.numpy as jnp
import numpy as np
from jax import lax
from jax.experimental import pallas as pl
from jax.experimental.pallas import tpu as pltpu
from jax.experimental.pallas import tpu_sc as plsc

_EPS = 1e-5


def _fps_body(npoint, xyz_ref, c_ref):
    x = xyz_ref[:, 0, :]
    y = xyz_ref[:, 1, :]
    z = xyz_ref[:, 2, :]
    B, N = x.shape
    iota = lax.broadcasted_iota(jnp.int32, (B, N), 1)
    slot = lax.broadcasted_iota(jnp.int32, (1, npoint), 1)

    def body(i, carry):
        dist, far, ax, ay, az = carry
        sel = iota == far
        cx = jnp.sum(jnp.where(sel, x, 0.0), axis=1, keepdims=True)
        cy = jnp.sum(jnp.where(sel, y, 0.0), axis=1, keepdims=True)
        cz = jnp.sum(jnp.where(sel, z, 0.0), axis=1, keepdims=True)
        hit = slot == i
        ax = jnp.where(hit, cx, ax)
        ay = jnp.where(hit, cy, ay)
        az = jnp.where(hit, cz, az)
        dx = x - cx
        dy = y - cy
        dz = z - cz
        d = dx * dx + dy * dy + dz * dz
        dist = jnp.minimum(dist, d)
        m = jnp.max(dist, axis=1, keepdims=True)
        far = jnp.min(jnp.where(dist == m, iota, N), axis=1, keepdims=True)
        return dist, far, ax, ay, az

    zc = jnp.zeros((B, npoint), jnp.float32)
    _, _, ax, ay, az = lax.fori_loop(
        0, npoint, body,
        (jnp.full((B, N), 1e10, jnp.float32), jnp.zeros((B, 1), jnp.int32),
         zc, zc, zc))
    c_ref[:, 0, :] = ax
    c_ref[:, 1, :] = ay
    c_ref[:, 2, :] = az


def _fps(xyz, npoint):
    B, _, N = xyz.shape
    return pl.pallas_call(
        functools.partial(_fps_body, npoint),
        out_shape=jax.ShapeDtypeStruct((B, 3, npoint), jnp.float32),
    )(xyz)


def _bq_body(r2, nsample, xyz_ref, ct_ref, idx_ref):
    pts = xyz_ref[0]
    ct = ct_ref[0]
    S = ct.shape[0]
    N = pts.shape[1]
    cross = jnp.dot(ct, pts, preferred_element_type=jnp.float32)
    c2 = jnp.sum(ct * ct, axis=1, keepdims=True)
    p2 = jnp.sum(pts * pts, axis=0, keepdims=True)
    d = -2.0 * cross
    d = d + c2
    d = d + p2
    iota = lax.broadcasted_iota(jnp.int32, (S, N), 1)
    cand0 = jnp.where(d > r2, N, iota)
    slot = lax.broadcasted_iota(jnp.int32, (1, nsample), 1)

    def body(j, carry):
        cand, first, out = carry
        m = jnp.min(cand, axis=1, keepdims=True)
        first = jnp.where(j == 0, m, first)
        val = jnp.where(m == N, first, m)
        out = jnp.where(slot == j, val, out)
        cand = jnp.where(cand == m, N, cand)
        return cand, first, out

    _, _, out = lax.fori_loop(
        0, nsample, body,
        (cand0, jnp.zeros((S, 1), jnp.int32),
         jnp.zeros((S, nsample), jnp.int32)))
    idx_ref[0] = out


def _bq(radius, nsample, xyz, ct):
    B, _, N = xyz.shape
    S = ct.shape[1]
    r2 = np.float32(float(radius) ** 2)
    return pl.pallas_call(
        functools.partial(_bq_body, r2, nsample),
        grid=(B,),
        in_specs=[
            pl.BlockSpec((1, 3, N), lambda b: (b, 0, 0)),
            pl.BlockSpec((1, S, 3), lambda b: (b, 0, 0)),
        ],
        out_specs=pl.BlockSpec((1, S, nsample), lambda b: (b, 0, 0)),
        out_shape=jax.ShapeDtypeStruct((B, S, nsample), jnp.int32),
    )(xyz, ct)


def _bqbits_body(r2, xyz_ref, ct_ref, pk_ref, bits_ref):
    pts = xyz_ref[0]
    ct = ct_ref[0]
    cross = jnp.dot(ct, pts, preferred_element_type=jnp.float32)
    c2 = jnp.sum(ct * ct, axis=1, keepdims=True)
    p2 = jnp.sum(pts * pts, axis=0, keepdims=True)
    d = -2.0 * cross
    d = d + c2
    d = d + p2
    mask = (d <= r2).astype(jnp.float32)
    bits_ref[0] = jnp.dot(mask, pk_ref[...], preferred_element_type=jnp.float32)


def _bqbits(radius, xyz, ct):
    B, _, N = xyz.shape
    S = ct.shape[1]
    NW = N // 32
    r2 = np.float32(float(radius) ** 2)
    n = np.arange(N)
    pk = np.zeros((N, 2 * NW), np.float32)
    bit = n % 32
    lo = bit < 16
    pk[n[lo], n[lo] // 32] = (2.0 ** bit[lo]).astype(np.float32)
    pk[n[~lo], NW + n[~lo] // 32] = (2.0 ** (bit[~lo] - 16)).astype(np.float32)
    return pl.pallas_call(
        functools.partial(_bqbits_body, r2),
        grid=(B,),
        in_specs=[
            pl.BlockSpec((1, 3, N), lambda b: (b, 0, 0)),
            pl.BlockSpec((1, S, 3), lambda b: (b, 0, 0)),
            pl.BlockSpec((N, 2 * NW), lambda b: (0, 0)),
        ],
        out_specs=pl.BlockSpec((1, S, 2 * NW), lambda b: (b, 0, 0)),
        out_shape=jax.ShapeDtypeStruct((B, S, 2 * NW), jnp.float32),
    )(xyz, ct, jnp.asarray(pk))


def _sc_ballgather(xyz, c, bits, K):
    B, _, N = xyz.shape
    S = c.shape[2]
    R = S * K
    SH = S // 2
    NPT = SH * K
    NW = N // 32
    NG = NW // 16
    BW = 2 * NW
    shift = int(np.log2(K))
    mesh = plsc.VectorSubcoreMesh(core_axis_name="c", subcore_axis_name="s")
    xs = xyz[:, 0, :].reshape(B * N)
    ys = xyz[:, 1, :].reshape(B * N)
    zs = xyz[:, 2, :].reshape(B * N)
    cxs = c[:, 0, :].reshape(B * S)
    cys = c[:, 1, :].reshape(B * S)
    czs = c[:, 2, :].reshape(B * S)
    bitsl = bits.reshape(B * S * BW)

    @functools.partial(
        pl.kernel,
        out_type=jax.ShapeDtypeStruct((B * R * 4,), jnp.float32),
        mesh=mesh,
        compiler_params=pltpu.CompilerParams(needs_layout_passes=False),
        scratch_types=[
            pltpu.VMEM((N,), jnp.float32),
            pltpu.VMEM((N,), jnp.float32),
            pltpu.VMEM((N,), jnp.float32),
            pltpu.VMEM((SH,), jnp.float32),
            pltpu.VMEM((SH,), jnp.float32),
            pltpu.VMEM((SH,), jnp.float32),
            pltpu.VMEM((SH * BW,), jnp.float32),
            pltpu.VMEM((NPT,), jnp.int32),
            pltpu.VMEM((NPT * 4,), jnp.float32),
        ],
    )
    def k(x_h, y_h, z_h, cx_h, cy_h, cz_h, bits_h, og_h,
          xv, yv, zv, cxv, cyv, czv, bv, idxb, bg):
        wid = lax.axis_index("s") * 2 + lax.axis_index("c")
        b = wid // 2
        half = wid - 2 * b
        s0 = half * SH
        r0 = b * R + s0 * K
        pltpu.sync_copy(x_h.at[pl.ds(b * N, N)], xv)
        pltpu.sync_copy(y_h.at[pl.ds(b * N, N)], yv)
        pltpu.sync_copy(z_h.at[pl.ds(b * N, N)], zv)
        pltpu.sync_copy(cx_h.at[pl.ds(b * S + s0, SH)], cxv)
        pltpu.sync_copy(cy_h.at[pl.ds(b * S + s0, SH)], cyv)
        pltpu.sync_copy(cz_h.at[pl.ds(b * S + s0, SH)], czv)
        pltpu.sync_copy(bits_h.at[pl.ds((b * S + s0) * BW, SH * BW)], bv)
        lane = lax.iota(jnp.int32, 16)

        def row_body(rl, _):
            rowbase = rl * K
            bitbase = rl * BW

            def grp_cond(st):
                return jnp.logical_and(st[0] < NG, st[1] < K)

            def grp_body(st):
                g, cnt = st
                off = bitbase + g * 16
                lov = lax.convert_element_type(bv[pl.ds(off, 16)], jnp.int32)
                hiv = lax.convert_element_type(bv[pl.ds(off + NW, 16)],
                                               jnp.int32)
                w = jnp.bitwise_or(lov, jnp.left_shift(hiv, 16))
                v = w - (lax.shift_right_logical(w, 1) & 0x55555555)
                v = (v & 0x33333333) + (lax.shift_right_logical(v, 2)
                                        & 0x33333333)
                v = (v + lax.shift_right_logical(v, 4)) & 0x0F0F0F0F
                v = v + lax.shift_right_logical(v, 8)
                pc = (v + lax.shift_right_logical(v, 16)) & 0x3F
                csum = plsc.cumsum(pc)
                o = cnt + (csum - pc)
                nbase = (g * 16 + lane) * 32

                def ext_cond(st2):
                    return jnp.any(st2[0] != 0)

                def ext_body(st2):
                    ww, ext = st2
                    m = ww & (-ww)
                    f = lax.convert_element_type(m, jnp.float32)
                    e = (lax.shift_right_logical(
                        plsc.bitcast(f, jnp.int32), 23) & 0xFF) - 127
                    nz = ww != 0
                    pos = o + ext
                    plsc.store_scatter(idxb, [rowbase + pos], nbase + e,
                                       mask=jnp.logical_and(nz, pos < K))
                    return ww & (ww - 1), ext + nz.astype(jnp.int32)

                lax.while_loop(ext_cond, ext_body,
                               (w, jnp.zeros((16,), jnp.int32)))
                return g + 1, cnt + jnp.sum(pc)

            _, cnt = lax.while_loop(grp_cond, grp_body,
                                    (jnp.int32(0), jnp.int32(0)))
            f0 = plsc.load_gather(idxb, [jnp.zeros((16,), jnp.int32)
                                         + rowbase])
            for q in range(K // 16):
                sl = q * 16 + lane
                plsc.store_scatter(idxb, [rowbase + sl], f0, mask=sl >= cnt)
            return 0

        lax.fori_loop(0, SH, row_body, 0)

        zero16 = jnp.zeros((16,), jnp.float32)

        def body(g, _):
            base = g * 16
            flat = base + lane
            sloc = jnp.right_shift(flat, shift)
            iv = idxb[pl.ds(base, 16)]
            tgt = flat * 4
            plsc.store_scatter(bg, [tgt],
                               plsc.load_gather(xv, [iv])
                               - plsc.load_gather(cxv, [sloc]))
            plsc.store_scatter(bg, [tgt + 1],
                               plsc.load_gather(yv, [iv])
                               - plsc.load_gather(cyv, [sloc]))
            plsc.store_scatter(bg, [tgt + 2],
                               plsc.load_gather(zv, [iv])
                               - plsc.load_gather(czv, [sloc]))
            plsc.store_scatter(bg, [tgt + 3], zero16)
            return 0

        lax.fori_loop(0, NPT // 16, body, 0)
        pltpu.sync_copy(bg, og_h.at[pl.ds(r0 * 4, NPT * 4)])

    og = k(xs, ys, zs, cxs, cys, czs, bitsl)
    return og.reshape(B, R, 4)


def _sc_ballidx(bits, K):
    B, S, BW = bits.shape
    NW = BW // 2
    NG = NW // 16
    R = S * K
    SH = (B * S) // 32
    NPT = SH * K
    mesh = plsc.VectorSubcoreMesh(core_axis_name="c", subcore_axis_name="s")
    bitsl = bits.reshape(B * S * BW)

    @functools.partial(
        pl.kernel,
        out_type=jax.ShapeDtypeStruct((B * R,), jnp.int32),
        mesh=mesh,
        compiler_params=pltpu.CompilerParams(needs_layout_passes=False),
        scratch_types=[
            pltpu.VMEM((SH * BW,), jnp.float32),
            pltpu.VMEM((NPT,), jnp.int32),
        ],
    )
    def k(bits_h, oi_h, bv, idxb):
        wid = lax.axis_index("s") * 2 + lax.axis_index("c")
        row0 = wid * SH
        pltpu.sync_copy(bits_h.at[pl.ds(row0 * BW, SH * BW)], bv)
        lane = lax.iota(jnp.int32, 16)

        def row_body(rl, _):
            rowbase = rl * K
            bitbase = rl * BW

            def grp_cond(st):
                return jnp.logical_and(st[0] < NG, st[1] < K)

            def grp_body(st):
                g, cnt = st
                off = bitbase + g * 16
                lov = lax.convert_element_type(bv[pl.ds(off, 16)], jnp.int32)
                hiv = lax.convert_element_type(bv[pl.ds(off + NW, 16)],
                                               jnp.int32)
                w = jnp.bitwise_or(lov, jnp.left_shift(hiv, 16))
                v = w - (lax.shift_right_logical(w, 1) & 0x55555555)
                v = (v & 0x33333333) + (lax.shift_right_logical(v, 2)
                                        & 0x33333333)
                v = (v + lax.shift_right_logical(v, 4)) & 0x0F0F0F0F
                v = v + lax.shift_right_logical(v, 8)
                pc = (v + lax.shift_right_logical(v, 16)) & 0x3F
                csum = plsc.cumsum(pc)
                o = cnt + (csum - pc)
                nbase = (g * 16 + lane) * 32

                def ext_cond(st2):
                    return jnp.any(st2[0] != 0)

                def ext_body(st2):
                    ww, ext = st2
                    m = ww & (-ww)
                    f = lax.convert_element_type(m, jnp.float32)
                    e = (lax.shift_right_logical(
                        plsc.bitcast(f, jnp.int32), 23) & 0xFF) - 127
                    nz = ww != 0
                    pos = o + ext
                    plsc.store_scatter(idxb, [rowbase + pos], nbase + e,
                                       mask=jnp.logical_and(nz, pos < K))
                    return ww & (ww - 1), ext + nz.astype(jnp.int32)

                lax.while_loop(ext_cond, ext_body,
                               (w, jnp.zeros((16,), jnp.int32)))
                return g + 1, cnt + jnp.sum(pc)

            _, cnt = lax.while_loop(grp_cond, grp_body,
                                    (jnp.int32(0), jnp.int32(0)))
            f0 = plsc.load_gather(idxb, [jnp.zeros((16,), jnp.int32)
                                         + rowbase])
            for q in range(K // 16):
                sl = q * 16 + lane
                plsc.store_scatter(idxb, [rowbase + sl], f0, mask=sl >= cnt)
            return 0

        lax.fori_loop(0, SH, row_body, 0)
        pltpu.sync_copy(idxb, oi_h.at[pl.ds(row0 * K, NPT)])

    return k(bitsl).reshape(B, R)


def _mlp1_body(K, w1_ref, b1_ref, w2_ref, b2_ref, w3_ref, b3_ref,
               g_ref, out_ref):
    x4 = g_ref[0]
    h = jnp.dot(x4, w1_ref[...], preferred_element_type=jnp.float32) + b1_ref[...]
    h = jnp.maximum(h, 0.0)
    h = jnp.dot(h, w2_ref[...], preferred_element_type=jnp.float32) + b2_ref[...]
    h = jnp.maximum(h, 0.0)
    h = jnp.dot(h, w3_ref[...], preferred_element_type=jnp.float32) + b3_ref[...]
    h = jnp.maximum(h, 0.0)
    R, C = h.shape
    out_ref[0] = jnp.max(h.reshape(R // K, K, C), axis=1)


def _mlp1(g, K, w1, b1, w2, b2, w3, b3):
    B, R, _ = g.shape
    S = R // K
    C = w3.shape[1]
    wspec = lambda s: pl.BlockSpec(s, lambda b: tuple(0 for _ in s))
    return pl.pallas_call(
        functools.partial(_mlp1_body, K),
        grid=(B,),
        in_specs=[wspec(w1.shape), wspec(b1.shape), wspec(w2.shape),
                  wspec(b2.shape), wspec(w3.shape), wspec(b3.shape),
                  pl.BlockSpec((1, R, 4), lambda b: (b, 0, 0))],
        out_specs=pl.BlockSpec((1, S, C), lambda b: (b, 0, 0)),
        out_shape=jax.ShapeDtypeStruct((B, S, C), jnp.float32),
    )(w1, b1, w2, b2, w3, b3, g)


def _mlp2_body(K, SB_S, w1x_ref, w1f_ref, b1_ref, w2_ref, b2_ref,
               w3_ref, b3_ref, idx_ref, xyz_ref, f_ref, c_ref, out_ref):
    ids = idx_ref[0]
    xyzt = xyz_ref[0]
    feats = f_ref[0]
    N = xyzt.shape[0]
    R = ids.shape[0]
    onehot = (lax.broadcasted_iota(jnp.int32, (R, N), 1) == ids
              ).astype(jnp.float32)
    gx = jnp.dot(onehot, xyzt, preferred_element_type=jnp.float32)
    gf = jnp.dot(onehot, feats, preferred_element_type=jnp.float32)
    cc = c_ref[0]
    crep = jnp.broadcast_to(cc[:, None, :], (SB_S, K, 4)).reshape(R, 4)
    dxyz = gx - crep
    h = (jnp.dot(dxyz, w1x_ref[...], preferred_element_type=jnp.float32)
         + jnp.dot(gf, w1f_ref[...], preferred_element_type=jnp.float32)
         + b1_ref[...])
    h = jnp.maximum(h, 0.0)
    h = jnp.dot(h, w2_ref[...], preferred_element_type=jnp.float32) + b2_ref[...]
    h = jnp.maximum(h, 0.0)
    h = jnp.dot(h, w3_ref[...], preferred_element_type=jnp.float32) + b3_ref[...]
    h = jnp.maximum(h, 0.0)
    C = h.shape[1]
    out_ref[0] = jnp.max(h.reshape(SB_S, K, C), axis=1)


def _mlp2(idxf, xyzr, featsr, cr, K, w1x, w1f, b1, w2, b2, w3, b3):
    B, R = idxf.shape
    S = R // K
    N = xyzr.shape[1]
    Cf = featsr.shape[2]
    C = w3.shape[1]
    SB = 4
    SB_S = S // SB
    RB = SB_S * K
    idxr = idxf.reshape(B, R, 1)
    wspec = lambda s: pl.BlockSpec(s, lambda b, sb: tuple(0 for _ in s))
    return pl.pallas_call(
        functools.partial(_mlp2_body, K, SB_S),
        grid=(B, SB),
        in_specs=[wspec(w1x.shape), wspec(w1f.shape), wspec(b1.shape),
                  wspec(w2.shape), wspec(b2.shape), wspec(w3.shape),
                  wspec(b3.shape),
                  pl.BlockSpec((1, RB, 1), lambda b, sb: (b, sb, 0)),
                  pl.BlockSpec((1, N, 4), lambda b, sb: (b, 0, 0)),
                  pl.BlockSpec((1, N, Cf), lambda b, sb: (b, 0, 0)),
                  pl.BlockSpec((1, SB_S, 4), lambda b, sb: (b, sb, 0))],
        out_specs=pl.BlockSpec((1, SB_S, C), lambda b, sb: (b, sb, 0)),
        out_shape=jax.ShapeDtypeStruct((B, S, C), jnp.float32),
    )(w1x, w1f, b1, w2, b2, w3, b3, idxr, xyzr, featsr, cr)


def _sa3_body(w1x_ref, w1f_ref, b1_ref, w2_ref, b2_ref, w3_ref, b3_ref,
              f1w_ref, f1b_ref, f2w_ref, f2b_ref,
              xyz_ref, f_ref, l3_ref, x_ref):
    xyzp = xyz_ref[0]
    f = f_ref[0]
    h = (jnp.dot(xyzp, w1x_ref[...], preferred_element_type=jnp.float32)
         + jnp.dot(f, w1f_ref[...], preferred_element_type=jnp.float32)
         + b1_ref[...])
    h = jnp.maximum(h, 0.0)
    h = jnp.dot(h, w2_ref[...], preferred_element_type=jnp.float32) + b2_ref[...]
    h = jnp.maximum(h, 0.0)
    h = jnp.dot(h, w3_ref[...], preferred_element_type=jnp.float32) + b3_ref[...]
    h = jnp.maximum(h, 0.0)
    l3 = jnp.max(h, axis=0, keepdims=True)
    l3_ref[0] = l3
    y = jnp.dot(l3, f1w_ref[...], preferred_element_type=jnp.float32) + f1b_ref[...]
    y = jnp.maximum(y, 0.0)
    y = jnp.dot(y, f2w_ref[...], preferred_element_type=jnp.float32) + f2b_ref[...]
    y = jnp.maximum(y, 0.0)
    x_ref[0] = y


def _sa3_head(xyzr, featsr, w1x, w1f, b1, w2, b2, w3, b3,
              f1w, f1b, f2w, f2b):
    B, S, Cf = featsr.shape
    wspec = lambda s: pl.BlockSpec(s, lambda b: tuple(0 for _ in s))
    return pl.pallas_call(
        _sa3_body,
        grid=(B,),
        in_specs=[wspec(w1x.shape), wspec(w1f.shape), wspec(b1.shape),
                  wspec(w2.shape), wspec(b2.shape), wspec(w3.shape),
                  wspec(b3.shape), wspec(f1w.shape), wspec(f1b.shape),
                  wspec(f2w.shape), wspec(f2b.shape),
                  pl.BlockSpec((1, S, 4), lambda b: (b, 0, 0)),
                  pl.BlockSpec((1, S, Cf), lambda b: (b, 0, 0))],
        out_specs=[pl.BlockSpec((1, 1, 1024), lambda b: (b, 0, 0)),
                   pl.BlockSpec((1, 1, 256), lambda b: (b, 0, 0))],
        out_shape=[jax.ShapeDtypeStruct((B, 1, 1024), jnp.float32),
                   jax.ShapeDtypeStruct((B, 1, 256), jnp.float32)],
    )(w1x, w1f, b1, w2, b2, w3, b3, f1w, f1b, f2w, f2b, xyzr, featsr)


def _fold(p):
    s = p['g'] / jnp.sqrt(p['rv'] + _EPS)
    w = p['W'] * s[None, :]
    b = ((p['b'] - p['rm']) * s + p['be'])[None, :]
    return w, b


def _pad4(w3):
    return jnp.concatenate([w3, jnp.zeros((1, w3.shape[1]), w3.dtype)], axis=0)


def kernel(xyz, params):
    B, _, N = xyz.shape
    sa1 = [_fold(p) for p in params['sa1']]
    sa2 = [_fold(p) for p in params['sa2']]
    sa3 = [_fold(p) for p in params['sa3']]

    def _fold_fc(fc, bn):
        s = bn['g'] / jnp.sqrt(bn['rv'] + _EPS)
        w = fc['W'] * s[None, :]
        b = ((fc['b'] - bn['rm']) * s + bn['be'])[None, :]
        return w, b

    f1w, f1b = _fold_fc(params['fc1'], params['bn1'])
    f2w, f2b = _fold_fc(params['fc2'], params['bn2'])

    c1 = _fps(xyz, 512)
    c1t = jnp.transpose(c1, (0, 2, 1))
    bits1 = _bqbits(0.2, xyz, c1t)
    g1 = _sc_ballgather(xyz, c1, bits1, 32)
    l1 = _mlp1(g1, 32,
               _pad4(sa1[0][0]), sa1[0][1], sa1[1][0], sa1[1][1],
               sa1[2][0], sa1[2][1])

    c2 = _fps(c1, 128)
    c2t = jnp.transpose(c2, (0, 2, 1))
    bits2 = _bqbits(0.4, c1, c2t)
    idx2 = _sc_ballidx(bits2, 64)
    c1t4 = jnp.concatenate(
        [c1t, jnp.zeros((B, c1t.shape[1], 1), jnp.float32)], axis=2)
    c2t4 = jnp.concatenate(
        [c2t, jnp.zeros((B, c2t.shape[1], 1), jnp.float32)], axis=2)
    w1 = sa2[0][0]
    l2 = _mlp2(idx2, c1t4, l1, c2t4, 64,
               _pad4(w1[:3]), w1[3:], sa2[0][1],
               sa2[1][0], sa2[1][1], sa2[2][0], sa2[2][1])

    w1g = sa3[0][0]
    l3, x = _sa3_head(c2t4, l2,
                      _pad4(w1g[:3]), w1g[3:], sa3[0][1],
                      sa3[1][0], sa3[1][1], sa3[2][0], sa3[2][1],
                      f1w, f1b, f2w, f2b)
    return x.reshape(B, 256), l3.reshape(B, 1024, 1)
